# baseline (speedup 1.0000x reference)
import functools

import jax
import jax.numpy as jnp
from jax.experimental import pallas as pl
from jax.experimental.pallas import tpu as pltpu

B, N, NFEAT, NHID, NDIM = 2, 4096, 128, 32, 16

BM = 512
BI = 512
BJ = 512


def _encoder_body(x_ref, w_ref, adj_ref, h_ref, support_ref):
    i = pl.program_id(1)

    @pl.when(i == 0)
    def _():
        support_ref[...] = jax.lax.dot(
            x_ref[0], w_ref[...], preferred_element_type=jnp.float32)

    h_ref[0] = jax.lax.dot(
        adj_ref[0], support_ref[...], preferred_element_type=jnp.float32)


def _decoder_body(hi_ref, hj_ref, mu_ref, sigma_ref):
    hi = hi_ref[0]
    hj = hj_ref[0]
    dims = (((1,), (1,)), ((), ()))
    mu = jax.lax.dot_general(hi[:, :NDIM], hj[:, :NDIM], dims,
                             preferred_element_type=jnp.float32)
    sigma = jax.lax.dot_general(hi[:, NDIM:], hj[:, NDIM:], dims,
                                preferred_element_type=jnp.float32)
    mu_ref[0] = jnp.maximum(mu, 0.0)
    sigma_ref[0] = jnp.maximum(sigma, 0.0)


@functools.partial(jax.jit)
def kernel(x, adj, W):
    h = pl.pallas_call(
        _encoder_body,
        grid=(B, N // BM),
        in_specs=[
            pl.BlockSpec((1, N, NFEAT), lambda b, i: (b, 0, 0)),
            pl.BlockSpec((NFEAT, NHID), lambda b, i: (0, 0)),
            pl.BlockSpec((1, BM, N), lambda b, i: (b, i, 0)),
        ],
        out_specs=pl.BlockSpec((1, BM, NHID), lambda b, i: (b, i, 0)),
        out_shape=jax.ShapeDtypeStruct((B, N, NHID), jnp.float32),
        scratch_shapes=[pltpu.VMEM((N, NHID), jnp.float32)],
    )(x, W, adj)

    mu, sigma = pl.pallas_call(
        _decoder_body,
        grid=(B, N // BI, N // BJ),
        in_specs=[
            pl.BlockSpec((1, BI, NHID), lambda b, i, j: (b, i, 0)),
            pl.BlockSpec((1, BJ, NHID), lambda b, i, j: (b, j, 0)),
        ],
        out_specs=[
            pl.BlockSpec((1, BI, BJ), lambda b, i, j: (b, i, j)),
            pl.BlockSpec((1, BI, BJ), lambda b, i, j: (b, i, j)),
        ],
        out_shape=[
            jax.ShapeDtypeStruct((B, N, N), jnp.float32),
            jax.ShapeDtypeStruct((B, N, N), jnp.float32),
        ],
    )(h, h)

    return (mu, sigma, h)


# speedup vs baseline: 1.3846x; 1.3846x over previous
import functools

import jax
import jax.numpy as jnp
from jax.experimental import pallas as pl
from jax.experimental.pallas import tpu as pltpu

B, N, NFEAT, NHID, NDIM = 2, 4096, 128, 32, 16

BM = 1024
BI = 1024
BJ = 1024


def _encoder_body(x_ref, w_ref, adj_ref, h_ref, support_ref):
    i = pl.program_id(1)

    @pl.when(i == 0)
    def _():
        support_ref[...] = jax.lax.dot(
            x_ref[0], w_ref[...], preferred_element_type=jnp.float32)

    h_ref[0] = jax.lax.dot(
        adj_ref[0], support_ref[...], preferred_element_type=jnp.float32)


def _decoder_body(hi_ref, hj_ref, mu_ref, sigma_ref):
    hi = hi_ref[0]
    hj = hj_ref[0]
    dims = (((1,), (1,)), ((), ()))
    mu = jax.lax.dot_general(hi[:, :NDIM], hj[:, :NDIM], dims,
                             preferred_element_type=jnp.float32)
    sigma = jax.lax.dot_general(hi[:, NDIM:], hj[:, NDIM:], dims,
                                preferred_element_type=jnp.float32)
    mu_ref[0] = jnp.maximum(mu, 0.0)
    sigma_ref[0] = jnp.maximum(sigma, 0.0)


@functools.partial(jax.jit)
def kernel(x, adj, W):
    h = pl.pallas_call(
        _encoder_body,
        grid=(B, N // BM),
        in_specs=[
            pl.BlockSpec((1, N, NFEAT), lambda b, i: (b, 0, 0)),
            pl.BlockSpec((NFEAT, NHID), lambda b, i: (0, 0)),
            pl.BlockSpec((1, BM, N), lambda b, i: (b, i, 0)),
        ],
        out_specs=pl.BlockSpec((1, BM, NHID), lambda b, i: (b, i, 0)),
        out_shape=jax.ShapeDtypeStruct((B, N, NHID), jnp.float32),
        scratch_shapes=[pltpu.VMEM((N, NHID), jnp.float32)],
        compiler_params=pltpu.CompilerParams(
            dimension_semantics=("arbitrary", "arbitrary")),
    )(x, W, adj)

    mu, sigma = pl.pallas_call(
        _decoder_body,
        grid=(B, N // BI, N // BJ),
        in_specs=[
            pl.BlockSpec((1, BI, NHID), lambda b, i, j: (b, i, 0)),
            pl.BlockSpec((1, BJ, NHID), lambda b, i, j: (b, j, 0)),
        ],
        out_specs=[
            pl.BlockSpec((1, BI, BJ), lambda b, i, j: (b, i, j)),
            pl.BlockSpec((1, BI, BJ), lambda b, i, j: (b, i, j)),
        ],
        out_shape=[
            jax.ShapeDtypeStruct((B, N, N), jnp.float32),
            jax.ShapeDtypeStruct((B, N, N), jnp.float32),
        ],
        compiler_params=pltpu.CompilerParams(
            dimension_semantics=("parallel", "parallel", "parallel")),
    )(h, h)

    return (mu, sigma, h)


# speedup vs baseline: 1.4389x; 1.0392x over previous
import functools

import jax
import jax.numpy as jnp
from jax.experimental import pallas as pl
from jax.experimental.pallas import tpu as pltpu

B, N, NFEAT, NHID, NDIM = 2, 4096, 128, 32, 16

BM = 1024
BI = 512
BJ = 4096


def _encoder_body(x_ref, w_ref, adj_ref, h_ref, support_ref):
    i = pl.program_id(1)

    @pl.when(i == 0)
    def _():
        support_ref[...] = jax.lax.dot(
            x_ref[0], w_ref[...], preferred_element_type=jnp.float32)

    h_ref[0] = jax.lax.dot(
        adj_ref[0], support_ref[...], preferred_element_type=jnp.float32)


def _decoder_body(hi_ref, hj_ref, mu_ref, sigma_ref):
    hi = hi_ref[0]
    hj = hj_ref[0]
    dims = (((1,), (1,)), ((), ()))
    mu = jax.lax.dot_general(hi[:, :NDIM], hj[:, :NDIM], dims,
                             preferred_element_type=jnp.float32)
    sigma = jax.lax.dot_general(hi[:, NDIM:], hj[:, NDIM:], dims,
                                preferred_element_type=jnp.float32)
    mu_ref[0] = jnp.maximum(mu, 0.0)
    sigma_ref[0] = jnp.maximum(sigma, 0.0)


@functools.partial(jax.jit)
def kernel(x, adj, W):
    h = pl.pallas_call(
        _encoder_body,
        grid=(B, N // BM),
        in_specs=[
            pl.BlockSpec((1, N, NFEAT), lambda b, i: (b, 0, 0)),
            pl.BlockSpec((NFEAT, NHID), lambda b, i: (0, 0)),
            pl.BlockSpec((1, BM, N), lambda b, i: (b, i, 0)),
        ],
        out_specs=pl.BlockSpec((1, BM, NHID), lambda b, i: (b, i, 0)),
        out_shape=jax.ShapeDtypeStruct((B, N, NHID), jnp.float32),
        scratch_shapes=[pltpu.VMEM((N, NHID), jnp.float32)],
        compiler_params=pltpu.CompilerParams(
            dimension_semantics=("arbitrary", "arbitrary")),
    )(x, W, adj)

    mu, sigma = pl.pallas_call(
        _decoder_body,
        grid=(B, N // BI, N // BJ),
        in_specs=[
            pl.BlockSpec((1, BI, NHID), lambda b, i, j: (b, i, 0)),
            pl.BlockSpec((1, BJ, NHID), lambda b, i, j: (b, j, 0)),
        ],
        out_specs=[
            pl.BlockSpec((1, BI, BJ), lambda b, i, j: (b, i, j)),
            pl.BlockSpec((1, BI, BJ), lambda b, i, j: (b, i, j)),
        ],
        out_shape=[
            jax.ShapeDtypeStruct((B, N, N), jnp.float32),
            jax.ShapeDtypeStruct((B, N, N), jnp.float32),
        ],
        compiler_params=pltpu.CompilerParams(
            dimension_semantics=("parallel", "parallel", "parallel")),
    )(h, h)

    return (mu, sigma, h)


# speedup vs baseline: 1.4567x; 1.0124x over previous
import functools

import jax
import jax.numpy as jnp
from jax.experimental import pallas as pl
from jax.experimental.pallas import tpu as pltpu

B, N, NFEAT, NHID, NDIM = 2, 4096, 128, 32, 16

BM = 512
BI = 256
BJ = 4096


def _encoder_body(x_ref, w_ref, adj_ref, h_ref, support_ref):
    i = pl.program_id(1)

    @pl.when(i == 0)
    def _():
        support_ref[...] = jax.lax.dot(
            x_ref[0], w_ref[...], preferred_element_type=jnp.float32)

    h_ref[0] = jax.lax.dot(
        adj_ref[0], support_ref[...], preferred_element_type=jnp.float32)


def _decoder_body(hi_ref, hj_ref, mu_ref, sigma_ref):
    hi = hi_ref[0]
    hj = hj_ref[0]
    dims = (((1,), (1,)), ((), ()))
    mu = jax.lax.dot_general(hi[:, :NDIM], hj[:, :NDIM], dims,
                             preferred_element_type=jnp.float32)
    sigma = jax.lax.dot_general(hi[:, NDIM:], hj[:, NDIM:], dims,
                                preferred_element_type=jnp.float32)
    mu_ref[0] = jnp.maximum(mu, 0.0)
    sigma_ref[0] = jnp.maximum(sigma, 0.0)


@functools.partial(jax.jit)
def kernel(x, adj, W):
    h = pl.pallas_call(
        _encoder_body,
        grid=(B, N // BM),
        in_specs=[
            pl.BlockSpec((1, N, NFEAT), lambda b, i: (b, 0, 0)),
            pl.BlockSpec((NFEAT, NHID), lambda b, i: (0, 0)),
            pl.BlockSpec((1, BM, N), lambda b, i: (b, i, 0)),
        ],
        out_specs=pl.BlockSpec((1, BM, NHID), lambda b, i: (b, i, 0)),
        out_shape=jax.ShapeDtypeStruct((B, N, NHID), jnp.float32),
        scratch_shapes=[pltpu.VMEM((N, NHID), jnp.float32)],
        compiler_params=pltpu.CompilerParams(
            dimension_semantics=("arbitrary", "arbitrary")),
    )(x, W, adj)

    mu, sigma = pl.pallas_call(
        _decoder_body,
        grid=(B, N // BI, N // BJ),
        in_specs=[
            pl.BlockSpec((1, BI, NHID), lambda b, i, j: (b, i, 0)),
            pl.BlockSpec((1, BJ, NHID), lambda b, i, j: (b, j, 0)),
        ],
        out_specs=[
            pl.BlockSpec((1, BI, BJ), lambda b, i, j: (b, i, j)),
            pl.BlockSpec((1, BI, BJ), lambda b, i, j: (b, i, j)),
        ],
        out_shape=[
            jax.ShapeDtypeStruct((B, N, N), jnp.float32),
            jax.ShapeDtypeStruct((B, N, N), jnp.float32),
        ],
        compiler_params=pltpu.CompilerParams(
            dimension_semantics=("parallel", "parallel", "parallel")),
    )(h, h)

    return (mu, sigma, h)
